# Initial kernel scaffold; baseline (speedup 1.0000x reference)
import jax
import jax.numpy as jnp
from jax.experimental import pallas as pl


def kernel(x, edge_index, W1, b1, W2, b2):
    raise NotImplementedError("write your pallas kernel here")


# speedup vs baseline: 30.8399x; 30.8399x over previous
import functools

import jax
import jax.numpy as jnp
from jax import lax
from jax.experimental import pallas as pl
from jax.experimental.pallas import tpu as pltpu
from jax.experimental.pallas import tpu_sc as plsc


_N_CORES = 2
_N_SUBCORES = 16
_N_WORKERS = _N_CORES * _N_SUBCORES


def _make_agg(n_rows, d, e_per_w):
    chunk = 64 if d >= 128 else 128
    nchunk = e_per_w // chunk
    rows_per_tile = n_rows // _N_SUBCORES
    zcopies = rows_per_tile // chunk
    mesh = plsc.VectorSubcoreMesh(core_axis_name="c", subcore_axis_name="s")

    @functools.partial(
        pl.kernel,
        mesh=mesh,
        compiler_params=pltpu.CompilerParams(use_tc_tiling_on_sc=False),
        out_type=jax.ShapeDtypeStruct((_N_CORES, n_rows, d), jnp.float32),
        scratch_types=[
            pltpu.VMEM((chunk, d), jnp.float32),
            pltpu.VMEM((chunk, d), jnp.float32),
            pltpu.VMEM((e_per_w,), jnp.int32),
            pltpu.VMEM((e_per_w,), jnp.int32),
            pltpu.VMEM((chunk,), jnp.int32),
            pltpu.VMEM((chunk,), jnp.int32),
            pltpu.VMEM_SHARED((n_rows, d), jnp.float32),
            pltpu.SemaphoreType.DMA,
            pltpu.SemaphoreType.DMA,
        ],
    )
    def agg(g_hbm, src_hbm, dst_hbm, out_hbm,
            rows_a, rows_b, src_w, dst_w, da, db, acc, sem_a, sem_b):
        cid = lax.axis_index("c")
        sid = lax.axis_index("s")
        wid = sid * _N_CORES + cid
        base = wid * e_per_w
        my_row0 = sid * rows_per_tile

        def _zero_row(i, carry):
            for j in range(d // 16):
                rows_a[i, pl.ds(j * 16, 16)] = jnp.zeros((16,), jnp.float32)
            return carry

        lax.fori_loop(0, chunk, _zero_row, 0)
        for k in range(zcopies):
            pltpu.sync_copy(rows_a, acc.at[pl.ds(my_row0 + k * chunk, chunk)])
        plsc.subcore_barrier()

        pltpu.sync_copy(src_hbm.at[pl.ds(base, e_per_w)], src_w)
        pltpu.sync_copy(dst_hbm.at[pl.ds(base, e_per_w)], dst_w)

        def _load_dst(c, dref):
            for j in range(chunk // 16):
                dref[pl.ds(j * 16, 16)] = dst_w[pl.ds(c * chunk + j * 16, 16)]

        def _fire(c, rbuf, sem):
            pltpu.async_copy(g_hbm.at[src_w.at[pl.ds(c * chunk, chunk)]], rbuf, sem)

        def _drain(rbuf, dref, sem):
            pltpu.make_async_copy(
                g_hbm.at[src_w.at[pl.ds(0, chunk)]], rbuf, sem).wait()
            pltpu.sync_copy(rbuf, acc.at[dref], add=True)

        _load_dst(0, da)
        _fire(0, rows_a, sem_a)

        def _pair(t, carry):
            c0 = t * 2
            c1 = c0 + 1
            _load_dst(c1, db)
            _fire(c1, rows_b, sem_b)
            _drain(rows_a, da, sem_a)

            @pl.when(c0 + 2 < nchunk)
            def _():
                _load_dst(c0 + 2, da)
                _fire(c0 + 2, rows_a, sem_a)

            _drain(rows_b, db, sem_b)
            return carry

        lax.fori_loop(0, nchunk // 2, _pair, 0)

        plsc.subcore_barrier()
        pltpu.sync_copy(acc.at[pl.ds(my_row0, rows_per_tile)],
                        out_hbm.at[cid, pl.ds(my_row0, rows_per_tile)])

    return agg


_BLK = 1024


def _deg_inv(degp_blk):
    deg = degp_blk[0, :, 0:1] + degp_blk[1, :, 0:1] + 1.0
    return lax.rsqrt(deg)


def _tc1_body(x_ref, w1_ref, degp_ref, g_ref):
    dinv = _deg_inv(degp_ref[...])
    h = jnp.dot(x_ref[...], w1_ref[...],
                preferred_element_type=jnp.float32,
                precision=lax.Precision.HIGHEST)
    g_ref[...] = h * dinv


def _tc2_body(acc_ref, g_ref, degp_ref, w2_ref, b1_ref, g2_ref):
    dinv = _deg_inv(degp_ref[...])
    pre = dinv * (acc_ref[0] + acc_ref[1] + g_ref[...]) + b1_ref[...]
    mid = jnp.maximum(pre, 0.0)
    z = jnp.dot(mid, w2_ref[...],
                preferred_element_type=jnp.float32,
                precision=lax.Precision.HIGHEST)
    g2_ref[...] = z * dinv


def _tc3_body(acc_ref, g2_ref, degp_ref, b2_ref, out_ref):
    dinv = _deg_inv(degp_ref[...])
    out_ref[...] = dinv * (acc_ref[0] + acc_ref[1] + g2_ref[...]) + b2_ref[...]


def _row_spec(d):
    return pl.BlockSpec((_BLK, d), lambda i: (i, 0))


def _acc_spec(d):
    return pl.BlockSpec((_N_CORES, _BLK, d), lambda i: (0, i, 0))


def _full_spec(shape):
    return pl.BlockSpec(shape, lambda i: tuple(0 for _ in shape))


def _tc1(xp, w1, degp, np_rows):
    return pl.pallas_call(
        _tc1_body,
        grid=(np_rows // _BLK,),
        in_specs=[_row_spec(128), _full_spec((128, 128)), _acc_spec(16)],
        out_specs=_row_spec(128),
        out_shape=jax.ShapeDtypeStruct((np_rows, 128), jnp.float32),
    )(xp, w1, degp)


def _tc2(acc_a, g, degp, w2p, b1r, np_rows):
    return pl.pallas_call(
        _tc2_body,
        grid=(np_rows // _BLK,),
        in_specs=[_acc_spec(128), _row_spec(128), _acc_spec(16),
                  _full_spec((128, 16)), _full_spec((1, 128))],
        out_specs=_row_spec(16),
        out_shape=jax.ShapeDtypeStruct((np_rows, 16), jnp.float32),
    )(acc_a, g, degp, w2p, b1r)


def _tc3(acc_b, g2, degp, b2r, np_rows):
    return pl.pallas_call(
        _tc3_body,
        grid=(np_rows // _BLK,),
        in_specs=[_acc_spec(16), _row_spec(16), _acc_spec(16),
                  _full_spec((1, 16))],
        out_specs=_row_spec(16),
        out_shape=jax.ShapeDtypeStruct((np_rows, 16), jnp.float32),
    )(acc_b, g2, degp, b2r)


def kernel(x, edge_index, W1, b1, W2, b2):
    n = x.shape[0]
    e = edge_index.shape[1]
    np_rows = 10240
    e_per_w = -(-e // (_N_WORKERS * 256)) * 256
    ep = _N_WORKERS * e_per_w

    pad_e = ep - e
    dummy = n + (jnp.arange(pad_e, dtype=jnp.int32) % (np_rows - n))
    srcp = jnp.concatenate([edge_index[0], dummy])
    dstp = jnp.concatenate([edge_index[1], dummy])

    xp = jnp.zeros((np_rows, 128), jnp.float32).at[:n].set(x)
    w2p = jnp.zeros((128, 16), jnp.float32).at[:, : W2.shape[1]].set(W2)
    b1r = b1.reshape(1, 128)
    b2r = jnp.zeros((1, 16), jnp.float32).at[0, : b2.shape[0]].set(b2)
    ones16 = jnp.ones((np_rows, 16), jnp.float32)

    agg16 = _make_agg(np_rows, 16, e_per_w)
    agg128 = _make_agg(np_rows, 128, e_per_w)

    degp = agg16(ones16, srcp, dstp)
    g = _tc1(xp, W1, degp, np_rows)
    acc_a = agg128(g, srcp, dstp)
    g2 = _tc2(acc_a, g, degp, w2p, b1r, np_rows)
    acc_b = agg16(g2, srcp, dstp)
    out16 = _tc3(acc_b, g2, degp, b2r, np_rows)
    return out16[:n, : W2.shape[1]]


# speedup vs baseline: 35.3082x; 1.1449x over previous
import functools

import jax
import jax.numpy as jnp
from jax import lax
from jax.experimental import pallas as pl
from jax.experimental.pallas import tpu as pltpu
from jax.experimental.pallas import tpu_sc as plsc

_N_CORES = 2
_N_SUBCORES = 16
_N_WORKERS = _N_CORES * _N_SUBCORES
_CHUNK = 128


def _make_agg(table_rows, acc_rows, d, e_per_w):
    nchunk = e_per_w // _CHUNK
    rows_per_tile = acc_rows // _N_SUBCORES
    zcopies = rows_per_tile // _CHUNK
    mesh = plsc.VectorSubcoreMesh(core_axis_name="c", subcore_axis_name="s")

    @functools.partial(
        pl.kernel,
        mesh=mesh,
        compiler_params=pltpu.CompilerParams(use_tc_tiling_on_sc=False),
        out_type=jax.ShapeDtypeStruct((_N_CORES, acc_rows, d), jnp.float32),
        scratch_types=[
            pltpu.VMEM((_CHUNK, d), jnp.float32),
            pltpu.VMEM((_CHUNK, d), jnp.float32),
            pltpu.VMEM((e_per_w,), jnp.int32),
            pltpu.VMEM((_CHUNK,), jnp.int32),
            pltpu.VMEM((_CHUNK,), jnp.int32),
            pltpu.VMEM_SHARED((acc_rows, d), jnp.float32),
            pltpu.SemaphoreType.DMA,
            pltpu.SemaphoreType.DMA,
            pltpu.SemaphoreType.DMA,
            pltpu.SemaphoreType.DMA,
        ],
    )
    def agg(g_hbm, src_hbm, dst_hbm, out_hbm,
            rows_a, rows_b, src_w, da, db, acc, sem_a, sem_b, sem_ia, sem_ib):
        cid = lax.axis_index("c")
        sid = lax.axis_index("s")
        wid = sid * _N_CORES + cid
        base = wid * e_per_w
        my_row0 = sid * rows_per_tile

        def _zero_row(i, carry):
            for j in range(d // 16):
                rows_a[i, pl.ds(j * 16, 16)] = jnp.zeros((16,), jnp.float32)
            return carry

        lax.fori_loop(0, _CHUNK, _zero_row, 0)
        for k in range(zcopies):
            pltpu.sync_copy(rows_a, acc.at[pl.ds(my_row0 + k * _CHUNK, _CHUNK)])
        plsc.subcore_barrier()

        pltpu.sync_copy(src_hbm.at[pl.ds(base, e_per_w)], src_w)

        def _fire_dst(c, dref, sem):
            pltpu.async_copy(dst_hbm.at[pl.ds(base + c * _CHUNK, _CHUNK)],
                             dref, sem)

        def _wait_dst(dref, sem):
            pltpu.make_async_copy(dst_hbm.at[pl.ds(base, _CHUNK)],
                                  dref, sem).wait()

        def _fire(c, rbuf, sem):
            pltpu.async_copy(g_hbm.at[src_w.at[pl.ds(c * _CHUNK, _CHUNK)]],
                             rbuf, sem)

        def _drain(rbuf, dref, sem):
            pltpu.make_async_copy(
                g_hbm.at[src_w.at[pl.ds(0, _CHUNK)]], rbuf, sem).wait()
            pltpu.sync_copy(rbuf, acc.at[dref], add=True)

        _fire_dst(0, da, sem_ia)
        _fire_dst(1, db, sem_ib)
        _fire(0, rows_a, sem_a)

        def _pair(t, carry):
            c0 = t * 2
            c1 = c0 + 1
            _fire(c1, rows_b, sem_b)
            _wait_dst(da, sem_ia)
            _drain(rows_a, da, sem_a)

            @pl.when(c0 + 2 < nchunk)
            def _():
                _fire_dst(c0 + 2, da, sem_ia)
                _fire(c0 + 2, rows_a, sem_a)

            _wait_dst(db, sem_ib)
            _drain(rows_b, db, sem_b)

            @pl.when(c1 + 2 < nchunk)
            def _():
                _fire_dst(c1 + 2, db, sem_ib)

            return carry

        lax.fori_loop(0, nchunk // 2, _pair, 0)

        plsc.subcore_barrier()
        pltpu.sync_copy(acc.at[pl.ds(my_row0, rows_per_tile)],
                        out_hbm.at[cid, pl.ds(my_row0, rows_per_tile)])

    return agg


def _make_deg(acc_rows, d, e_per_w):
    nchunk = e_per_w // _CHUNK
    rows_per_tile = acc_rows // _N_SUBCORES
    zcopies = rows_per_tile // _CHUNK
    mesh = plsc.VectorSubcoreMesh(core_axis_name="c", subcore_axis_name="s")

    @functools.partial(
        pl.kernel,
        mesh=mesh,
        compiler_params=pltpu.CompilerParams(use_tc_tiling_on_sc=False),
        out_type=jax.ShapeDtypeStruct((_N_CORES, acc_rows, d), jnp.float32),
        scratch_types=[
            pltpu.VMEM((_CHUNK, d), jnp.float32),
            pltpu.VMEM((_CHUNK,), jnp.int32),
            pltpu.VMEM((_CHUNK,), jnp.int32),
            pltpu.VMEM_SHARED((acc_rows, d), jnp.float32),
            pltpu.SemaphoreType.DMA,
            pltpu.SemaphoreType.DMA,
            pltpu.SemaphoreType.DMA,
            pltpu.SemaphoreType.DMA,
        ],
    )
    def deg(dst_hbm, out_hbm, ones_v, da, db, acc,
            sem_sa, sem_sb, sem_ia, sem_ib):
        cid = lax.axis_index("c")
        sid = lax.axis_index("s")
        wid = sid * _N_CORES + cid
        base = wid * e_per_w
        my_row0 = sid * rows_per_tile

        def _fill_row(i, carry):
            for j in range(d // 16):
                ones_v[i, pl.ds(j * 16, 16)] = jnp.zeros((16,), jnp.float32)
            return carry

        lax.fori_loop(0, _CHUNK, _fill_row, 0)
        for k in range(zcopies):
            pltpu.sync_copy(ones_v, acc.at[pl.ds(my_row0 + k * _CHUNK, _CHUNK)])
        plsc.subcore_barrier()

        def _ones_row(i, carry):
            for j in range(d // 16):
                ones_v[i, pl.ds(j * 16, 16)] = jnp.ones((16,), jnp.float32)
            return carry

        lax.fori_loop(0, _CHUNK, _ones_row, 0)

        def _fire_dst(c, dref, sem):
            pltpu.async_copy(dst_hbm.at[pl.ds(base + c * _CHUNK, _CHUNK)],
                             dref, sem)

        def _wait_dst(dref, sem):
            pltpu.make_async_copy(dst_hbm.at[pl.ds(base, _CHUNK)],
                                  dref, sem).wait()

        def _fire_scatter(dref, sem):
            pltpu.async_copy(ones_v, acc.at[dref], sem, add=True)

        def _wait_scatter(dref, sem):
            pltpu.make_async_copy(ones_v, acc.at[dref], sem).wait()

        _fire_dst(0, da, sem_ia)
        _fire_dst(1, db, sem_ib)

        def _pair(t, carry):
            c0 = t * 2
            c1 = c0 + 1
            _wait_dst(da, sem_ia)
            _fire_scatter(da, sem_sa)
            _wait_dst(db, sem_ib)
            _fire_scatter(db, sem_sb)
            _wait_scatter(da, sem_sa)

            @pl.when(c0 + 2 < nchunk)
            def _():
                _fire_dst(c0 + 2, da, sem_ia)

            _wait_scatter(db, sem_sb)

            @pl.when(c1 + 2 < nchunk)
            def _():
                _fire_dst(c1 + 2, db, sem_ib)

            return carry

        lax.fori_loop(0, nchunk // 2, _pair, 0)

        plsc.subcore_barrier()
        pltpu.sync_copy(acc.at[pl.ds(my_row0, rows_per_tile)],
                        out_hbm.at[cid, pl.ds(my_row0, rows_per_tile)])

    return deg


_BLK = 1000


def _deg_inv(degp_blk):
    deg = degp_blk[0, :, 0:1] + degp_blk[1, :, 0:1] + 1.0
    return lax.rsqrt(deg)


def _tc1_body(x_ref, w1_ref, degp_ref, g_ref):
    dinv = _deg_inv(degp_ref[...])
    h = jnp.dot(x_ref[...], w1_ref[...],
                preferred_element_type=jnp.float32,
                precision=lax.Precision.HIGHEST)
    g_ref[...] = h * dinv


def _tc2_body(acc_ref, g_ref, degp_ref, w2_ref, b1_ref, g2_ref):
    dinv = _deg_inv(degp_ref[...])
    pre = dinv * (acc_ref[0] + acc_ref[1] + g_ref[...]) + b1_ref[...]
    mid = jnp.maximum(pre, 0.0)
    z = jnp.dot(mid, w2_ref[...],
                preferred_element_type=jnp.float32,
                precision=lax.Precision.HIGHEST)
    g2_ref[...] = z * dinv


def _tc3_body(acc_ref, g2_ref, degp_ref, b2_ref, out_ref):
    dinv = _deg_inv(degp_ref[...])
    out_ref[...] = dinv * (acc_ref[0] + acc_ref[1] + g2_ref[...]) + b2_ref[...]


def _row_spec(d):
    return pl.BlockSpec((_BLK, d), lambda i: (i, 0))


def _acc_spec(d):
    return pl.BlockSpec((_N_CORES, _BLK, d), lambda i: (0, i, 0))


def _full_spec(shape):
    return pl.BlockSpec(shape, lambda i: tuple(0 for _ in shape))


def _tc1(x, w1, degp, n):
    return pl.pallas_call(
        _tc1_body,
        grid=(n // _BLK,),
        in_specs=[_row_spec(128), _full_spec((128, 128)), _acc_spec(16)],
        out_specs=_row_spec(128),
        out_shape=jax.ShapeDtypeStruct((n, 128), jnp.float32),
    )(x, w1, degp)


def _tc2(acc_a, g, degp, w2p, b1r, n):
    return pl.pallas_call(
        _tc2_body,
        grid=(n // _BLK,),
        in_specs=[_acc_spec(128), _row_spec(128), _acc_spec(16),
                  _full_spec((128, 16)), _full_spec((1, 128))],
        out_specs=_row_spec(16),
        out_shape=jax.ShapeDtypeStruct((n, 16), jnp.float32),
    )(acc_a, g, degp, w2p, b1r)


def _tc3(acc_b, g2, degp, b2r, n):
    return pl.pallas_call(
        _tc3_body,
        grid=(n // _BLK,),
        in_specs=[_acc_spec(16), _row_spec(16), _acc_spec(16),
                  _full_spec((1, 16))],
        out_specs=_row_spec(16),
        out_shape=jax.ShapeDtypeStruct((n, 16), jnp.float32),
    )(acc_b, g2, degp, b2r)


def kernel(x, edge_index, W1, b1, W2, b2):
    n = x.shape[0]
    e = edge_index.shape[1]
    acc_rows = 10240
    e_per_w = -(-e // (_N_WORKERS * 2 * _CHUNK)) * (2 * _CHUNK)
    ep = _N_WORKERS * e_per_w

    pad_e = ep - e
    idx = jnp.arange(pad_e, dtype=jnp.int32)
    srcp = jnp.concatenate([edge_index[0], idx % n])
    dstp = jnp.concatenate([edge_index[1], n + idx % (acc_rows - n)])

    w2p = jnp.zeros((128, 16), jnp.float32).at[:, : W2.shape[1]].set(W2)
    b1r = b1.reshape(1, 128)
    b2r = jnp.zeros((1, 16), jnp.float32).at[0, : b2.shape[0]].set(b2)

    degp = _make_deg(acc_rows, 16, e_per_w)(dstp)
    g = _tc1(x, W1, degp, n)
    acc_a = _make_agg(n, acc_rows, 128, e_per_w)(g, srcp, dstp)
    g2 = _tc2(acc_a, g, degp, w2p, b1r, n)
    acc_b = _make_agg(n, acc_rows, 16, e_per_w)(g2, srcp, dstp)
    out16 = _tc3(acc_b, g2, degp, b2r, n)
    return out16[:, : W2.shape[1]]
